# baseline (speedup 1.0000x reference)
import jax
import jax.numpy as jnp
from jax.experimental import pallas as pl


def _copy_kernel(x_ref, o_ref):
    o_ref[...] = x_ref[...]


def kernel(xyz_gradient_accum, denom, opacity_accum, max_radii2D, viewspace_grad, opacity, radii, update_idx):
    grad_norm = jnp.linalg.norm(viewspace_grad, axis=-1, keepdims=True)
    a = xyz_gradient_accum.at[update_idx].add(grad_norm)
    b = denom.at[update_idx].add(jnp.ones_like(grad_norm))
    c = opacity_accum.at[update_idx].add(jax.nn.sigmoid(opacity))
    d = max_radii2D.at[update_idx].max(radii)
    out = jnp.concatenate([a, b, c, d[:, None]], axis=1)
    flat = out.reshape(8, 1, -1)
    res = pl.pallas_call(
        _copy_kernel,
        out_shape=jax.ShapeDtypeStruct(flat.shape, flat.dtype),
        grid=(8,),
        in_specs=[pl.BlockSpec((1, 1, flat.shape[2]), lambda i: (i, 0, 0))],
        out_specs=pl.BlockSpec((1, 1, flat.shape[2]), lambda i: (i, 0, 0)),
    )(flat)
    return res.reshape(out.shape)


# speedup vs baseline: 3.7463x; 3.7463x over previous
import functools

import jax
import jax.numpy as jnp
from jax import lax
from jax.experimental import pallas as pl
from jax.experimental.pallas import tpu as pltpu
from jax.experimental.pallas import tpu_sc as plsc

NP = 1_000_000
M = 500_000
MP = 524_288
ROWS = MP // 128
RPT = ROWS // 32
SC_HALF = NP // 2
OWN = 31_264
OWN_LAST = NP - 31 * OWN
DUM = 16
HR = SC_HALF // 2
NDUM = 2048


def _prep_body(gx_ref, gy_ref, op_ref, n_ref, s_ref):
    gx = gx_ref[...]
    gy = gy_ref[...]
    n_ref[...] = jnp.sqrt(gx * gx + gy * gy)
    x = op_ref[...]
    s_ref[...] = 1.0 / (1.0 + jnp.exp(-x))


def _prep(gx, gy, op):
    return pl.pallas_call(
        _prep_body,
        out_shape=[jax.ShapeDtypeStruct((ROWS, 128), jnp.float32)] * 2,
        grid=(8,),
        in_specs=[pl.BlockSpec((ROWS // 8, 128), lambda i: (i, 0))] * 3,
        out_specs=[pl.BlockSpec((ROWS // 8, 128), lambda i: (i, 0))] * 2,
    )(gx, gy, op)


_mesh = plsc.VectorSubcoreMesh(core_axis_name="c", subcore_axis_name="s")


@functools.partial(
    pl.kernel,
    out_type=[jax.ShapeDtypeStruct((NP,), jnp.float32)] * 4,
    mesh=_mesh,
    compiler_params=pltpu.CompilerParams(needs_layout_passes=False),
    scratch_types=[
        pltpu.VMEM((RPT, 128), jnp.int32),
        pltpu.VMEM((2 * RPT, 128), jnp.int32),
        pltpu.VMEM((RPT, 128), jnp.float32),
        pltpu.VMEM((128,), jnp.float32),
        pltpu.VMEM((OWN + DUM,), jnp.float32),
        pltpu.VMEM((16, 128), jnp.int32),
        pltpu.VMEM((16, 128), jnp.float32),
        pltpu.VMEM((2000,), jnp.float32),
        pltpu.VMEM_SHARED((HR + NDUM,), jnp.float32),
    ],
)
def _sc_scatter(idx_hbm, norm_hbm, sig_hbm, radii_hbm,
                xyz_in, den_in, opa_in, mr_in,
                out_xyz, out_den, out_opa, out_mr,
                idx_v, ridx_v, val_v, ones_v, mr_v,
                sidx_v, srad_v, stage_v, plane):
    c = lax.axis_index("c")
    s = lax.axis_index("s")
    wid = c * 16 + s
    sc_base = c * SC_HALF
    lane = lax.iota(jnp.int32, 16)

    for j in range(8):
        ones_v[pl.ds(j * 16, 16)] = jnp.full((16,), 1.0, jnp.float32)

    for h in range(2):
        range_base = sc_base + h * HR

        for seg in range(2):
            rowbase = s * (2 * RPT) + seg * RPT
            pltpu.sync_copy(idx_hbm.at[pl.ds(rowbase, RPT), :], idx_v)

            def _route(i, _, seg=seg, range_base=range_base):
                r = i // 8
                col = (i % 8) * 16
                v = idx_v[r, pl.ds(col, 16)]
                l = v - range_base
                owned = (l >= 0) & (l < HR)
                dummy = HR + (i % 128) * 16 + lane
                ridx_v[seg * RPT + r, pl.ds(col, 16)] = jnp.where(
                    owned, l, dummy)
                return 0

            lax.fori_loop(0, RPT * 8, _route, 0)

        for src_in, out_hbm, vals_hbm in ((xyz_in, out_xyz, norm_hbm),
                                          (den_in, out_den, None),
                                          (opa_in, out_opa, sig_hbm)):
            for m in range(8):
                k = m * 16 + s

                @pl.when(k < 125)
                def _(k=k, src_in=src_in, range_base=range_base):
                    off = k * 2000
                    pltpu.sync_copy(src_in.at[pl.ds(range_base + off, 2000)],
                                    stage_v)
                    pltpu.sync_copy(stage_v, plane.at[pl.ds(off, 2000)])

            plsc.subcore_barrier()

            for seg in range(2):
                rowbase = s * (2 * RPT) + seg * RPT
                if vals_hbm is not None:
                    pltpu.sync_copy(vals_hbm.at[pl.ds(rowbase, RPT), :],
                                    val_v)

                    def _scat(r, _, seg=seg):
                        pltpu.sync_copy(val_v.at[r],
                                        plane.at[ridx_v.at[seg * RPT + r]],
                                        add=True)
                        return 0
                else:
                    def _scat(r, _, seg=seg):
                        pltpu.sync_copy(ones_v,
                                        plane.at[ridx_v.at[seg * RPT + r]],
                                        add=True)
                        return 0

                lax.fori_loop(0, RPT, _scat, 0)

            plsc.subcore_barrier()

            for m in range(8):
                k = m * 16 + s

                @pl.when(k < 125)
                def _(k=k, out_hbm=out_hbm, range_base=range_base):
                    off = k * 2000
                    pltpu.sync_copy(plane.at[pl.ds(off, 2000)], stage_v)
                    pltpu.sync_copy(stage_v,
                                    out_hbm.at[pl.ds(range_base + off, 2000)])

            plsc.subcore_barrier()

    own_base = wid * OWN
    own_len = jnp.where(wid == 31, OWN_LAST, OWN)

    @pl.when(wid < 31)
    def _():
        pltpu.sync_copy(mr_in.at[pl.ds(own_base, OWN)], mr_v.at[pl.ds(0, OWN)])

    @pl.when(wid == 31)
    def _():
        pltpu.sync_copy(mr_in.at[pl.ds(own_base, OWN_LAST)],
                        mr_v.at[pl.ds(0, OWN_LAST)])

    mr_v[pl.ds(OWN, 16)] = jnp.zeros((16,), jnp.float32)

    def _chunk(k, _):
        pltpu.sync_copy(idx_hbm.at[pl.ds(k * 16, 16), :], sidx_v)
        pltpu.sync_copy(radii_hbm.at[pl.ds(k * 16, 16), :], srad_v)

        def _vec(i, _):
            r = i // 8
            col = (i % 8) * 16
            v = sidx_v[r, pl.ds(col, 16)]
            rad = srad_v[r, pl.ds(col, 16)]
            l = v - own_base
            owned = (l >= 0) & (l < own_len)
            routed = jnp.where(owned, l, OWN + lane)
            vals = jnp.where(owned, rad, -1.0)

            cur = plsc.load_gather(mr_v, [routed])
            newv = jnp.maximum(cur, vals)
            plsc.store_scatter(mr_v, [routed], newv, mask=owned)
            cur2 = plsc.load_gather(mr_v, [routed])
            conflict = owned & (cur2 < vals)
            pend0 = jnp.where(conflict, jnp.int32(1), jnp.int32(0))

            @pl.when(jnp.max(pend0) > 0)
            def _():
                def _retry(_, pend):
                    pm = pend > 0
                    c1 = plsc.load_gather(mr_v, [routed])
                    n1 = jnp.maximum(c1, vals)
                    plsc.store_scatter(mr_v, [routed], n1, mask=pm)
                    c2 = plsc.load_gather(mr_v, [routed])
                    still = pm & (c2 < vals)
                    return jnp.where(still, jnp.int32(1), jnp.int32(0))

                lax.fori_loop(0, 16, _retry, pend0)

            return 0

        lax.fori_loop(0, 128, _vec, 0)
        return 0

    lax.fori_loop(0, ROWS // 16, _chunk, 0)

    @pl.when(wid < 31)
    def _():
        pltpu.sync_copy(mr_v.at[pl.ds(0, OWN)], out_mr.at[pl.ds(own_base, OWN)])

    @pl.when(wid == 31)
    def _():
        pltpu.sync_copy(mr_v.at[pl.ds(0, OWN_LAST)],
                        out_mr.at[pl.ds(own_base, OWN_LAST)])


def kernel(xyz_gradient_accum, denom, opacity_accum, max_radii2D,
           viewspace_grad, opacity, radii, update_idx):
    f32 = jnp.float32
    padn = MP - M
    zpad = jnp.zeros((padn,), f32)
    pad_idx = NP + (jnp.arange(padn, dtype=jnp.int32) % 16)
    idx_p = jnp.concatenate([update_idx.astype(jnp.int32), pad_idx])
    gx = jnp.concatenate([viewspace_grad[:, 0], zpad]).reshape(ROWS, 128)
    gy = jnp.concatenate([viewspace_grad[:, 1], zpad]).reshape(ROWS, 128)
    op = jnp.concatenate([opacity[:, 0], zpad]).reshape(ROWS, 128)
    rd = jnp.concatenate([radii, zpad - 1.0]).reshape(ROWS, 128)
    norm_p, sig_p = _prep(gx, gy, op)
    o0, o1, o2, o3 = _sc_scatter(
        idx_p.reshape(ROWS, 128), norm_p, sig_p, rd,
        xyz_gradient_accum.reshape(-1), denom.reshape(-1),
        opacity_accum.reshape(-1), max_radii2D)
    return jnp.stack([o0, o1, o2, o3], axis=1)
